# Initial kernel scaffold; baseline (speedup 1.0000x reference)
import jax
import jax.numpy as jnp
from jax.experimental import pallas as pl


def kernel(x, W):
    raise NotImplementedError("write your pallas kernel here")


# speedup vs baseline: 1.4584x; 1.4584x over previous
import functools

import jax
import jax.numpy as jnp
from jax import lax
from jax.experimental import pallas as pl
from jax.experimental.pallas import tpu as pltpu
from jax.experimental.pallas import tpu_sc as plsc

_NC = 2
_NS = 16
_NW = _NC * _NS


@functools.cache
def _make_gather(total: int, vocab: int, dim: int, chunk: int):
    b_per_w = total // _NW
    n_chunks = b_per_w // chunk
    mesh = plsc.VectorSubcoreMesh(core_axis_name="c", subcore_axis_name="s")

    @functools.partial(
        pl.kernel,
        mesh=mesh,
        out_type=jax.ShapeDtypeStruct((total, dim), jnp.float32),
        scratch_types=[
            pltpu.VMEM((chunk,), jnp.int32),
            pltpu.VMEM((chunk, dim), jnp.float32),
            pltpu.SemaphoreType.DMA,
        ],
        compiler_params=pltpu.CompilerParams(use_tc_tiling_on_sc=False),
    )
    def gather_kernel(idx_hbm, table_hbm, out_hbm, idx_v, rows_v, sem):
        wid = lax.axis_index("s") * _NC + lax.axis_index("c")
        base = wid * b_per_w

        def body(c, carry):
            off = base + c * chunk
            pltpu.sync_copy(idx_hbm.at[pl.ds(off, chunk)], idx_v)
            pltpu.async_copy(table_hbm.at[idx_v], rows_v, sem).wait()
            pltpu.sync_copy(rows_v, out_hbm.at[pl.ds(off, chunk)])
            return carry

        lax.fori_loop(0, n_chunks, body, 0)

    return gather_kernel


def kernel(x, W):
    B, L = x.shape
    vocab, dim = W.shape
    total = B * L
    flat_idx = x.reshape(total)
    out = _make_gather(total, vocab, dim, 1024)(flat_idx, W)
    return out.reshape(B, L, dim)


# speedup vs baseline: 1.4940x; 1.0244x over previous
import functools

import jax
import jax.numpy as jnp
from jax import lax
from jax.experimental import pallas as pl
from jax.experimental.pallas import tpu as pltpu
from jax.experimental.pallas import tpu_sc as plsc

_NC = 2
_NS = 16
_NW = _NC * _NS
_NBUF = 2


@functools.cache
def _make_gather(total: int, vocab: int, dim: int, chunk: int):
    b_per_w = total // _NW
    n_chunks = b_per_w // chunk
    n_groups = n_chunks // _NBUF
    mesh = plsc.VectorSubcoreMesh(core_axis_name="c", subcore_axis_name="s")

    @functools.partial(
        pl.kernel,
        mesh=mesh,
        out_type=jax.ShapeDtypeStruct((total, dim), jnp.float32),
        scratch_types=[
            pltpu.VMEM((chunk,), jnp.int32),
            pltpu.VMEM((chunk,), jnp.int32),
            pltpu.VMEM((chunk, dim), jnp.float32),
            pltpu.VMEM((chunk, dim), jnp.float32),
            pltpu.SemaphoreType.DMA,
            pltpu.SemaphoreType.DMA,
            pltpu.SemaphoreType.DMA,
            pltpu.SemaphoreType.DMA,
            pltpu.SemaphoreType.DMA,
        ],
        compiler_params=pltpu.CompilerParams(use_tc_tiling_on_sc=False),
    )
    def gather_kernel(idx_hbm, table_hbm, out_hbm, idx0, idx1, rows0, rows1,
                      isem0, isem1, osem0, osem1, gsem):
        idx_v = [idx0, idx1]
        rows_v = [rows0, rows1]
        isem = [isem0, isem1]
        osem = [osem0, osem1]
        wid = lax.axis_index("s") * _NC + lax.axis_index("c")
        base = wid * b_per_w

        for b in range(_NBUF):
            pltpu.async_copy(
                idx_hbm.at[pl.ds(base + b * chunk, chunk)], idx_v[b], isem[b])

        def body(g, carry):
            for b in range(_NBUF):
                c = g * _NBUF + b
                off = base + c * chunk

                pltpu.make_async_copy(
                    idx_hbm.at[pl.ds(off, chunk)], idx_v[b], isem[b]).wait()

                @pl.when(g > 0)
                def _():
                    pltpu.make_async_copy(
                        rows_v[b], out_hbm.at[pl.ds(off, chunk)], osem[b]).wait()

                pltpu.async_copy(table_hbm.at[idx_v[b]], rows_v[b], gsem).wait()

                @pl.when(c + _NBUF < n_chunks)
                def _():
                    pltpu.async_copy(
                        idx_hbm.at[pl.ds(off + _NBUF * chunk, chunk)],
                        idx_v[b], isem[b])

                pltpu.async_copy(
                    rows_v[b], out_hbm.at[pl.ds(off, chunk)], osem[b])
            return carry

        lax.fori_loop(0, n_groups, body, 0)

        for b in range(_NBUF):
            off = base + (n_chunks - _NBUF + b) * chunk
            pltpu.make_async_copy(
                rows_v[b], out_hbm.at[pl.ds(off, chunk)], osem[b]).wait()

    return gather_kernel


def kernel(x, W):
    B, L = x.shape
    vocab, dim = W.shape
    total = B * L
    flat_idx = x.reshape(total)
    out = _make_gather(total, vocab, dim, 1600)(flat_idx, W)
    return out.reshape(B, L, dim)


# speedup vs baseline: 1.5012x; 1.0048x over previous
import functools

import jax
import jax.numpy as jnp
from jax import lax
from jax.experimental import pallas as pl
from jax.experimental.pallas import tpu as pltpu
from jax.experimental.pallas import tpu_sc as plsc

_NC = 2
_NS = 16
_NW = _NC * _NS
_NBUF = 2


@functools.cache
def _make_gather(total: int, vocab: int, dim: int, chunk: int):
    b_per_w = total // _NW
    n_chunks = b_per_w // chunk
    n_groups = n_chunks // _NBUF
    mesh = plsc.VectorSubcoreMesh(core_axis_name="c", subcore_axis_name="s")

    @functools.partial(
        pl.kernel,
        mesh=mesh,
        out_type=jax.ShapeDtypeStruct((total, dim), jnp.float32),
        scratch_types=[
            pltpu.VMEM((chunk,), jnp.int32),
            pltpu.VMEM((chunk,), jnp.int32),
            pltpu.VMEM((chunk, dim), jnp.float32),
            pltpu.VMEM((chunk, dim), jnp.float32),
            pltpu.SemaphoreType.DMA,
            pltpu.SemaphoreType.DMA,
            pltpu.SemaphoreType.DMA,
            pltpu.SemaphoreType.DMA,
            pltpu.SemaphoreType.DMA,
            pltpu.SemaphoreType.DMA,
        ],
        compiler_params=pltpu.CompilerParams(use_tc_tiling_on_sc=False),
    )
    def gather_kernel(idx_hbm, table_hbm, out_hbm, idx0, idx1, rows0, rows1,
                      isem0, isem1, osem0, osem1, gsem0, gsem1):
        idx_v = [idx0, idx1]
        rows_v = [rows0, rows1]
        isem = [isem0, isem1]
        osem = [osem0, osem1]
        gsem = [gsem0, gsem1]
        wid = lax.axis_index("s") * _NC + lax.axis_index("c")
        base = wid * b_per_w

        def issue_gather(b):
            pltpu.async_copy(table_hbm.at[idx_v[b]], rows_v[b], gsem[b])

        def wait_gather(b):
            pltpu.make_async_copy(table_hbm.at[idx_v[b]], rows_v[b],
                                  gsem[b]).wait()

        def wait_idx(b, off):
            pltpu.make_async_copy(
                idx_hbm.at[pl.ds(off, chunk)], idx_v[b], isem[b]).wait()

        def drain_out(b, off):
            pltpu.make_async_copy(
                rows_v[b], out_hbm.at[pl.ds(off, chunk)], osem[b]).wait()

        for b in range(_NBUF):
            pltpu.async_copy(
                idx_hbm.at[pl.ds(base + b * chunk, chunk)], idx_v[b], isem[b])
        wait_idx(0, base)
        issue_gather(0)

        def body(g, carry):
            for s in range(_NBUF):
                c = g * _NBUF + s
                off = base + c * chunk
                nb = 1 - s

                def launch_next():
                    wait_idx(nb, off + chunk)
                    issue_gather(nb)

                if s == 0:
                    @pl.when(g > 0)
                    def _():
                        drain_out(nb, off - chunk)
                    launch_next()
                else:
                    @pl.when(g < n_groups - 1)
                    def _():
                        drain_out(nb, off - chunk)
                        launch_next()

                wait_gather(s)

                @pl.when(c + _NBUF < n_chunks)
                def _():
                    pltpu.async_copy(
                        idx_hbm.at[pl.ds(off + _NBUF * chunk, chunk)],
                        idx_v[s], isem[s])

                pltpu.async_copy(
                    rows_v[s], out_hbm.at[pl.ds(off, chunk)], osem[s])
            return carry

        lax.fori_loop(0, n_groups, body, 0)

        for b in range(_NBUF):
            off = base + (n_chunks - _NBUF + b) * chunk
            drain_out(b, off)

    return gather_kernel


def kernel(x, W):
    B, L = x.shape
    vocab, dim = W.shape
    total = B * L
    flat_idx = x.reshape(total)
    out = _make_gather(total, vocab, dim, 1600)(flat_idx, W)
    return out.reshape(B, L, dim)
